# baseline (speedup 1.0000x reference)
import functools

import jax
import jax.numpy as jnp
import numpy as np
from jax import lax
from jax.experimental import pallas as pl
from jax.experimental.pallas import tpu as pltpu
from jax.experimental.pallas import tpu_sc as plsc

_NC, _NS = 2, 16
_NW = _NC * _NS
_NG = 64


def _edge_mlp(attr, w1, b1, w2, b2):
    e = attr.shape[0]
    dout = w2.shape[1]
    be = 4000

    def body(a_ref, w1_ref, b1_ref, w2_ref, b2_ref, o_ref):
        a = a_ref[...]
        h = jnp.maximum(
            jnp.dot(a, w1_ref[...], preferred_element_type=jnp.float32)
            + b1_ref[...], 0.0)
        o_ref[...] = (
            jnp.dot(h, w2_ref[...], preferred_element_type=jnp.float32)
            + b2_ref[...]).astype(jnp.bfloat16)

    return pl.pallas_call(
        body,
        grid=(e // be,),
        in_specs=[
            pl.BlockSpec((be, attr.shape[1]), lambda i: (i, 0)),
            pl.BlockSpec(w1.shape, lambda i: (0, 0)),
            pl.BlockSpec(b1.shape, lambda i: (0, 0)),
            pl.BlockSpec(w2.shape, lambda i: (0, 0)),
            pl.BlockSpec(b2.shape, lambda i: (0, 0)),
        ],
        out_specs=pl.BlockSpec((be, dout), lambda i: (i, 0)),
        out_shape=jax.ShapeDtypeStruct((e, dout), jnp.bfloat16),
    )(attr, w1, b1, w2, b2)


def _sc_msgpass(x, ee, eic):
    n, d = x.shape
    e = eic.shape[0] * eic.shape[2]
    ew = e // _NW
    c_sz = eic.shape[2]
    nch = ew // c_sz
    rpt, zc = 624, 16

    mesh = plsc.VectorSubcoreMesh(
        core_axis_name="c", subcore_axis_name="s",
        num_cores=_NC, num_subcores=_NS)

    @functools.partial(
        pl.kernel,
        out_type=jax.ShapeDtypeStruct((_NC, n, d), jnp.float32),
        mesh=mesh,
        scratch_types=[
            pltpu.VMEM((3, 2, c_sz), jnp.int32),
            [pltpu.VMEM((c_sz * d // 2,), jnp.int32)] * 3,
            pltpu.VMEM((3, c_sz, d), jnp.float32),
            pltpu.VMEM((zc, d), jnp.float32),
            pltpu.VMEM_SHARED((n, d), jnp.float32),
            [pltpu.SemaphoreType.DMA] * 3,
            [pltpu.SemaphoreType.DMA] * 3,
            [pltpu.SemaphoreType.DMA] * 3,
            [pltpu.SemaphoreType.DMA] * 3,
            pltpu.SemaphoreType.DMA,
        ],
    )
    def k(x_hbm, ee_hbm, ei_hbm, out_hbm,
          idx_v, ee_v, rows_v, zero_v, agg_sh, es, gs, isem, ss, dsem):
        ci = lax.axis_index("c")
        si = lax.axis_index("s")
        wid = si * _NC + ci

        def zrow(i, _):
            for j in range(d // 16):
                zero_v[i, pl.ds(j * 16, 16)] = jnp.zeros((16,), jnp.float32)
            return 0

        lax.fori_loop(0, zc, zrow, 0)
        for z in range(rpt // zc):
            pltpu.async_copy(zero_v,
                             agg_sh.at[pl.ds(si * rpt + z * zc, zc)], dsem)

        @pl.when(si == _NS - 1)
        def _():
            pltpu.async_copy(zero_v,
                             agg_sh.at[pl.ds(_NS * rpt, n - _NS * rpt)], dsem)

        for z in range(rpt // zc):
            pltpu.make_async_copy(
                zero_v, agg_sh.at[pl.ds(z * zc, zc)], dsem).wait()

        @pl.when(si == _NS - 1)
        def _():
            pltpu.make_async_copy(
                zero_v, agg_sh.at[pl.ds(0, zc)], dsem).wait()

        plsc.subcore_barrier()

        base = wid * ew
        base_g = wid * nch

        def slot(t, b, prefetch_next=True, prefetch_idx=True):
            nxt, n2 = (b + 1) % 3, (b + 2) % 3
            if prefetch_next:
                pltpu.make_async_copy(ei_hbm.at[base_g],
                                      idx_v.at[nxt], isem[nxt]).wait()
                pltpu.async_copy(
                    ee_hbm.at[pl.ds((base + (t + 1) * c_sz) * (d // 2),
                                    c_sz * d // 2)],
                    ee_v[nxt], es[nxt])
                pltpu.async_copy(x_hbm.at[idx_v.at[nxt, 0]],
                                 rows_v.at[nxt], gs[nxt])
            pltpu.make_async_copy(ee_hbm.at[pl.ds(0, c_sz * d // 2)],
                                  ee_v[b], es[b]).wait()
            pltpu.make_async_copy(x_hbm.at[pl.ds(0, c_sz)],
                                  rows_v.at[b], gs[b]).wait()
            eeb = ee_v[b]

            def msg(i, _):
                for g2 in range(d // 32):
                    ew = eeb[pl.ds(i * (d // 2) + g2 * 16, 16)]
                    ea = lax.bitcast_convert_type(
                        lax.shift_left(ew, 16), jnp.float32)
                    eb = lax.bitcast_convert_type(
                        jnp.bitwise_and(ew, jnp.int32(-65536)), jnp.float32)
                    sa = pl.ds((2 * g2) * 16, 16)
                    sb = pl.ds((2 * g2 + 1) * 16, 16)
                    rows_v[b, i, sa] = jnp.maximum(rows_v[b, i, sa] + ea, 0.0)
                    rows_v[b, i, sb] = jnp.maximum(rows_v[b, i, sb] + eb, 0.0)
                return 0

            lax.fori_loop(0, c_sz, msg, 0)
            pltpu.async_copy(rows_v.at[b], agg_sh.at[idx_v.at[b, 1]],
                             ss[b], add=True)
            pltpu.make_async_copy(rows_v.at[n2],
                                  agg_sh.at[pl.ds(0, c_sz)], ss[n2]).wait()
            if prefetch_idx:
                pltpu.async_copy(ei_hbm.at[base_g + t + 2],
                                 idx_v.at[n2], isem[n2])

        pltpu.sync_copy(ei_hbm.at[base_g], idx_v.at[0])
        pltpu.async_copy(ei_hbm.at[base_g + 1], idx_v.at[1], isem[1])
        pltpu.async_copy(ee_hbm.at[pl.ds(base * (d // 2), c_sz * d // 2)],
                         ee_v[0], es[0])
        pltpu.async_copy(x_hbm.at[idx_v.at[0, 0]], rows_v.at[0], gs[0])
        pltpu.async_copy(x_hbm.at[pl.ds(0, c_sz)], rows_v.at[2], ss[2])

        def triple(i, _):
            t0 = i * 3
            for db in (0, 1, 2):
                slot(t0 + db, db)
            return 0

        lax.fori_loop(0, nch // 3, triple, 0)
        slot(nch - 2, (nch - 2) % 3, prefetch_idx=False)
        slot(nch - 1, (nch - 1) % 3, prefetch_next=False, prefetch_idx=False)
        b_last = (nch - 1) % 3
        pltpu.make_async_copy(rows_v.at[b_last],
                              agg_sh.at[pl.ds(0, c_sz)], ss[b_last]).wait()

        plsc.subcore_barrier()

        for z in range(3):
            row0 = si * rpt + z * 208
            pltpu.sync_copy(agg_sh.at[pl.ds(row0, 208)],
                            out_hbm.at[ci, pl.ds(row0, 208)])

        @pl.when(si == _NS - 1)
        def _():
            pltpu.sync_copy(agg_sh.at[pl.ds(_NS * rpt, n - _NS * rpt)],
                            out_hbm.at[ci, pl.ds(_NS * rpt, n - _NS * rpt)])

    return k(x, ee, eic)


def _node_stats(xin, agg0, agg1, w1, b1, w2, b2, eps11):
    n, din = xin.shape
    d2 = w2.shape[1]
    nb = 1000
    nblk = n // nb

    def body(x_ref, a0_ref, a1_ref, w1_ref, b1_ref, w2_ref, b2_ref, eps_ref,
             y_ref, st_ref):
        i = pl.program_id(0)
        h = (x_ref[...] * (1.0 + eps_ref[0, 0])
             + a0_ref[...] + a1_ref[...])
        t = jnp.maximum(
            jnp.dot(h, w1_ref[...], preferred_element_type=jnp.float32)
            + b1_ref[...], 0.0)
        y = jnp.maximum(
            jnp.dot(t, w2_ref[...], preferred_element_type=jnp.float32)
            + b2_ref[...], 0.0)
        y_ref[...] = y

        @pl.when(i == 0)
        def _():
            st_ref[...] = jnp.zeros_like(st_ref)

        st_ref[0:1, :] += jnp.sum(y, axis=0, keepdims=True)
        st_ref[1:2, :] += jnp.sum(y * y, axis=0, keepdims=True)

    return pl.pallas_call(
        body,
        grid=(nblk,),
        in_specs=[
            pl.BlockSpec((nb, din), lambda i: (i, 0)),
            pl.BlockSpec((nb, din), lambda i: (i, 0)),
            pl.BlockSpec((nb, din), lambda i: (i, 0)),
            pl.BlockSpec(w1.shape, lambda i: (0, 0)),
            pl.BlockSpec(b1.shape, lambda i: (0, 0)),
            pl.BlockSpec(w2.shape, lambda i: (0, 0)),
            pl.BlockSpec(b2.shape, lambda i: (0, 0)),
            pl.BlockSpec((1, 1), lambda i: (0, 0)),
        ],
        out_specs=[
            pl.BlockSpec((nb, d2), lambda i: (i, 0)),
            pl.BlockSpec((8, d2), lambda i: (0, 0)),
        ],
        out_shape=[
            jax.ShapeDtypeStruct((n, d2), jnp.float32),
            jax.ShapeDtypeStruct((8, d2), jnp.float32),
        ],
    )(xin, agg0, agg1, w1, b1, w2, b2, eps11)


def _bn_apply(y, st, g, b):
    n, d2 = y.shape
    nb = 2000

    def body(y_ref, st_ref, g_ref, b_ref, o_ref):
        mean = st_ref[0:1, :] / n
        var = st_ref[1:2, :] / n - mean * mean
        o_ref[...] = ((y_ref[...] - mean) * lax.rsqrt(var + 1e-5)
                      * g_ref[...] + b_ref[...])

    return pl.pallas_call(
        body,
        grid=(n // nb,),
        in_specs=[
            pl.BlockSpec((nb, d2), lambda i: (i, 0)),
            pl.BlockSpec((8, d2), lambda i: (0, 0)),
            pl.BlockSpec((1, d2), lambda i: (0, 0)),
            pl.BlockSpec((1, d2), lambda i: (0, 0)),
        ],
        out_specs=pl.BlockSpec((nb, d2), lambda i: (i, 0)),
        out_shape=jax.ShapeDtypeStruct((n, d2), jnp.float32),
    )(y, st, g, b)


def _pool_head(x1, x2, x3, x4, batch_row,
               w1, b1, w2, b2, w3, b3, w4, b4):
    n, d = x1.shape
    nb = 2000
    nblk = n // nb

    def body(b_ref, x1_ref, x2_ref, x3_ref, x4_ref,
             w1_ref, b1_ref, w2_ref, b2_ref, w3_ref, b3_ref, w4_ref, b4_ref,
             o_ref, acc_ref, cnt_ref):
        i = pl.program_id(0)

        @pl.when(i == 0)
        def _():
            acc_ref[...] = jnp.zeros_like(acc_ref)
            cnt_ref[...] = jnp.zeros_like(cnt_ref)

        seg = lax.broadcasted_iota(jnp.int32, (_NG, nb), 0)
        oh = (seg == b_ref[...].reshape(1, nb)).astype(jnp.float32)
        xcat = jnp.concatenate(
            [x1_ref[...], x2_ref[...], x3_ref[...], x4_ref[...]], axis=1)
        acc_ref[...] += lax.dot_general(
            oh, xcat, (((1,), (0,)), ((), ())),
            preferred_element_type=jnp.float32)
        cnt_ref[...] += jnp.sum(oh, axis=1, keepdims=True)

        @pl.when(i == nblk - 1)
        def _():
            pooled = acc_ref[...] / jnp.maximum(cnt_ref[...], 1.0)
            h1 = jnp.maximum(
                jnp.dot(pooled, w1_ref[...],
                        preferred_element_type=jnp.float32) + b1_ref[...], 0.0)
            h2 = jnp.maximum(
                jnp.dot(h1, w2_ref[...],
                        preferred_element_type=jnp.float32) + b2_ref[...], 0.0)
            h3 = jnp.maximum(
                jnp.dot(h2, w3_ref[...],
                        preferred_element_type=jnp.float32) + b3_ref[...], 0.0)
            z = (jnp.dot(h3, w4_ref[...],
                         preferred_element_type=jnp.float32) + b4_ref[...])
            m = jnp.max(z, axis=1, keepdims=True)
            lse = m + jnp.log(jnp.sum(jnp.exp(z - m), axis=1, keepdims=True))
            o_ref[...] = z - lse

    return pl.pallas_call(
        body,
        grid=(nblk,),
        in_specs=[
            pl.BlockSpec((1, 1, nb), lambda i: (i, 0, 0)),
            pl.BlockSpec((nb, d), lambda i: (i, 0)),
            pl.BlockSpec((nb, d), lambda i: (i, 0)),
            pl.BlockSpec((nb, d), lambda i: (i, 0)),
            pl.BlockSpec((nb, d), lambda i: (i, 0)),
            pl.BlockSpec(w1.shape, lambda i: (0, 0)),
            pl.BlockSpec(b1.shape, lambda i: (0, 0)),
            pl.BlockSpec(w2.shape, lambda i: (0, 0)),
            pl.BlockSpec(b2.shape, lambda i: (0, 0)),
            pl.BlockSpec(w3.shape, lambda i: (0, 0)),
            pl.BlockSpec(b3.shape, lambda i: (0, 0)),
            pl.BlockSpec(w4.shape, lambda i: (0, 0)),
            pl.BlockSpec(b4.shape, lambda i: (0, 0)),
        ],
        out_specs=pl.BlockSpec((_NG, 2), lambda i: (0, 0)),
        out_shape=jax.ShapeDtypeStruct((_NG, 2), jnp.float32),
        scratch_shapes=[
            pltpu.VMEM((_NG, 4 * d), jnp.float32),
            pltpu.VMEM((_NG, 1), jnp.float32),
        ],
    )(batch_row, x1, x2, x3, x4, w1, b1, w2, b2, w3, b3, w4, b4)


_PSI = np.array([(p // 32) * 32 + 2 * (p % 16) + ((p // 16) % 2)
                 for p in range(128)])
_PSI4 = np.concatenate([_PSI + 128 * l for l in range(4)])


def _pad2(w, r, c):
    return jnp.pad(w, ((0, r - w.shape[0]), (0, c - w.shape[1])))


def _padb(b, c):
    return jnp.pad(b, (0, c - b.shape[0])).reshape(1, -1)


def kernel(x, edge_index, edge_attr, batch, params):
    p = params
    c_sz = 80
    eic = jnp.stack([edge_index[0].reshape(-1, c_sz),
                     edge_index[1].reshape(-1, c_sz)], axis=1)

    c1 = p["conv1"]
    ee1 = _edge_mlp(edge_attr,
                    _pad2(c1["be1"]["W"], 3, 16), _padb(c1["be1"]["b"], 16),
                    _pad2(c1["be2"]["W"], 16, 128), _padb(c1["be2"]["b"], 128))
    ees = [
        _edge_mlp(edge_attr, cv["be1"]["W"], cv["be1"]["b"].reshape(1, -1),
                  cv["be2"]["W"], cv["be2"]["b"].reshape(1, -1))
        for cv in (p["conv2"], p["conv3"], p["conv4"])
    ]

    x128 = jnp.pad(x, ((0, 0), (0, 128 - x.shape[1])))[:, _PSI]

    def layer(xin, cv, ee, bn, pad_in):
        eew = lax.bitcast_convert_type(
            ee.reshape(-1, ee.shape[1] // 2, 2), jnp.int32).reshape(-1)
        ag = _sc_msgpass(xin, eew, eic)
        if pad_in:
            w1 = _pad2(cv["m1"]["W"], 128, 16)[_PSI, :]
            b1 = _padb(cv["m1"]["b"], 16)
            w2 = _pad2(cv["m2"]["W"], 16, 128)[:, _PSI]
        else:
            w1 = cv["m1"]["W"][_PSI, :]
            b1 = cv["m1"]["b"].reshape(1, -1)
            w2 = cv["m2"]["W"][:, _PSI]
        b2 = cv["m2"]["b"][_PSI].reshape(1, -1)
        y, st = _node_stats(xin, ag[0], ag[1], w1, b1, w2, b2,
                            cv["eps"].reshape(1, 1))
        return _bn_apply(y, st, bn["g"][_PSI].reshape(1, -1),
                         bn["b"][_PSI].reshape(1, -1))

    x1r = layer(x128, p["conv1"], ee1, p["bn1"], True)
    x2r = layer(x1r, p["conv2"], ees[0], p["bn2"], False)
    x3r = layer(x2r, p["conv3"], ees[1], p["bn3"], False)
    x4r = layer(x3r, p["conv4"], ees[2], p["bn4"], False)

    return _pool_head(
        x1r, x2r, x3r, x4r, batch.reshape(-1, 1, 2000),
        p["fc1"]["W"][_PSI4, :], p["fc1"]["b"].reshape(1, -1),
        p["fc2"]["W"], p["fc2"]["b"].reshape(1, -1),
        p["fc3"]["W"], p["fc3"]["b"].reshape(1, -1),
        p["fc4"]["W"], p["fc4"]["b"].reshape(1, -1))


# speedup vs baseline: 2.6110x; 2.6110x over previous
import functools

import jax
import jax.numpy as jnp
from jax import lax
from jax.experimental import pallas as pl
from jax.experimental.pallas import tpu as pltpu
from jax.experimental.pallas import tpu_sc as plsc

_NC, _NS = 2, 16
_NW = _NC * _NS
_NG = 64


def _edge_mlp(attr, w1, b1, w2, b2):
    e = attr.shape[0]
    dout = w2.shape[1]
    be = 4000

    def body(a_ref, w1_ref, b1_ref, w2_ref, b2_ref, o_ref):
        a = a_ref[...]
        h = jnp.maximum(
            jnp.dot(a, w1_ref[...], preferred_element_type=jnp.float32)
            + b1_ref[...], 0.0)
        y = (jnp.dot(h.astype(jnp.bfloat16),
                     w2_ref[...].astype(jnp.bfloat16),
                     preferred_element_type=jnp.float32) + b2_ref[...])
        o_ref[...] = pltpu.bitcast(y.astype(jnp.bfloat16), jnp.int32)

    return pl.pallas_call(
        body,
        grid=(e // be,),
        in_specs=[
            pl.BlockSpec((be, attr.shape[1]), lambda i: (i, 0)),
            pl.BlockSpec(w1.shape, lambda i: (0, 0)),
            pl.BlockSpec(b1.shape, lambda i: (0, 0)),
            pl.BlockSpec(w2.shape, lambda i: (0, 0)),
            pl.BlockSpec(b2.shape, lambda i: (0, 0)),
        ],
        out_specs=pl.BlockSpec((be // 2, dout), lambda i: (i, 0)),
        out_shape=jax.ShapeDtypeStruct((e // 2, dout), jnp.int32),
    )(attr, w1, b1, w2, b2)


def _sc_msgpass(x, eew, eic):
    n, d = x.shape
    e = eic.shape[0] * eic.shape[2]
    ew = e // _NW
    c_sz = eic.shape[2]
    nch = ew // c_sz
    rpt, zc = 624, 16

    mesh = plsc.VectorSubcoreMesh(
        core_axis_name="c", subcore_axis_name="s",
        num_cores=_NC, num_subcores=_NS)

    @functools.partial(
        pl.kernel,
        out_type=jax.ShapeDtypeStruct((_NC, n, d), jnp.float32),
        mesh=mesh,
        scratch_types=[
            pltpu.VMEM((2, 2, c_sz), jnp.int32),
            pltpu.VMEM((2, c_sz // 2, d), jnp.int32),
            pltpu.VMEM((2, c_sz, d), jnp.float32),
            pltpu.VMEM((zc, d), jnp.float32),
            pltpu.VMEM_SHARED((n, d), jnp.float32),
            [pltpu.SemaphoreType.DMA] * 2,
            [pltpu.SemaphoreType.DMA] * 2,
            pltpu.SemaphoreType.DMA,
        ],
    )
    def k(x_hbm, ee_hbm, ei_hbm, out_hbm,
          idx_v, ee_v, rows_v, zero_v, agg_sh, es, gs, dsem):
        ci = lax.axis_index("c")
        si = lax.axis_index("s")
        wid = si * _NC + ci

        def zrow(i, _):
            for j in range(d // 16):
                zero_v[i, pl.ds(j * 16, 16)] = jnp.zeros((16,), jnp.float32)
            return 0

        lax.fori_loop(0, zc, zrow, 0)
        for z in range(rpt // zc):
            pltpu.async_copy(zero_v,
                             agg_sh.at[pl.ds(si * rpt + z * zc, zc)], dsem)

        @pl.when(si == _NS - 1)
        def _():
            pltpu.async_copy(zero_v,
                             agg_sh.at[pl.ds(_NS * rpt, n - _NS * rpt)], dsem)

        for z in range(rpt // zc):
            pltpu.make_async_copy(
                zero_v, agg_sh.at[pl.ds(z * zc, zc)], dsem).wait()

        @pl.when(si == _NS - 1)
        def _():
            pltpu.make_async_copy(
                zero_v, agg_sh.at[pl.ds(0, zc)], dsem).wait()

        plsc.subcore_barrier()

        base = wid * ew
        base_g = wid * nch

        def start_fetch(t, b):
            pltpu.async_copy(
                ee_hbm.at[pl.ds(wid * (ew // 2) + t * (c_sz // 2),
                                c_sz // 2)],
                ee_v.at[b], es[b])
            pltpu.async_copy(x_hbm.at[idx_v.at[b, 0]], rows_v.at[b], gs[b])

        def finish_chunk(b):
            pltpu.make_async_copy(ee_hbm.at[pl.ds(0, c_sz // 2)],
                                  ee_v.at[b], es[b]).wait()
            pltpu.make_async_copy(x_hbm.at[pl.ds(0, c_sz)],
                                  rows_v.at[b], gs[b]).wait()

            def msg(r, _):
                for j in range(d // 16):
                    sl = pl.ds(j * 16, 16)
                    w = ee_v[b, r, sl]
                    ea = lax.bitcast_convert_type(
                        lax.shift_left(w, 16), jnp.float32)
                    eb = lax.bitcast_convert_type(
                        jnp.bitwise_and(w, jnp.int32(-65536)), jnp.float32)
                    rows_v[b, 2 * r, sl] = jnp.maximum(
                        rows_v[b, 2 * r, sl] + ea, 0.0)
                    rows_v[b, 2 * r + 1, sl] = jnp.maximum(
                        rows_v[b, 2 * r + 1, sl] + eb, 0.0)
                return 0

            lax.fori_loop(0, c_sz // 2, msg, 0)
            pltpu.sync_copy(rows_v.at[b], agg_sh.at[idx_v.at[b, 1]], add=True)

        pltpu.sync_copy(ei_hbm.at[base_g], idx_v.at[0])
        start_fetch(0, 0)

        def pair(i, _):
            t0 = i * 2
            for db in (0, 1):
                tn = t0 + db + 1
                nb = 1 - db
                pltpu.sync_copy(ei_hbm.at[base_g + tn], idx_v.at[nb])
                start_fetch(tn, nb)
                finish_chunk(db)
            return 0

        lax.fori_loop(0, (nch - 1) // 2, pair, 0)
        finish_chunk((nch - 1) % 2)

        plsc.subcore_barrier()

        for z in range(rpt // 208):
            row0 = si * rpt + z * 208
            pltpu.sync_copy(agg_sh.at[pl.ds(row0, 208)],
                            out_hbm.at[ci, pl.ds(row0, 208)])

        @pl.when(si == _NS - 1)
        def _():
            pltpu.sync_copy(agg_sh.at[pl.ds(_NS * rpt, n - _NS * rpt)],
                            out_hbm.at[ci, pl.ds(_NS * rpt, n - _NS * rpt)])

    return k(x, eew, eic)


def _node_fused(xin, agg0, agg1, w1, b1, w2, b2, eps11, g, b):
    n, din = xin.shape
    d2 = w2.shape[1]
    nb = 1000
    nblk = n // nb

    def body(x_ref, a0_ref, a1_ref, w1_ref, b1_ref, w2_ref, b2_ref, eps_ref,
             g_ref, b_ref, o_ref, y_ref, st_ref):
        ph = pl.program_id(0)
        i = pl.program_id(1)

        @pl.when(ph == 0)
        def _():
            h = (x_ref[...] * (1.0 + eps_ref[0, 0])
                 + a0_ref[...] + a1_ref[...])
            t = jnp.maximum(
                jnp.dot(h, w1_ref[...], preferred_element_type=jnp.float32)
                + b1_ref[...], 0.0)
            y = jnp.maximum(
                jnp.dot(t, w2_ref[...], preferred_element_type=jnp.float32)
                + b2_ref[...], 0.0)
            y_ref[pl.ds(i * nb, nb), :] = y

            @pl.when(i == 0)
            def _():
                st_ref[...] = jnp.zeros_like(st_ref)

            st_ref[0:1, :] += jnp.sum(y, axis=0, keepdims=True)
            st_ref[1:2, :] += jnp.sum(y * y, axis=0, keepdims=True)

        @pl.when(ph == 1)
        def _():
            mean = st_ref[0:1, :] / n
            var = st_ref[1:2, :] / n - mean * mean
            o_ref[...] = ((y_ref[pl.ds(i * nb, nb), :] - mean)
                          * lax.rsqrt(var + 1e-5) * g_ref[...] + b_ref[...])

    return pl.pallas_call(
        body,
        grid=(2, nblk),
        in_specs=[
            pl.BlockSpec((nb, din), lambda ph, i: (i * (1 - ph), 0)),
            pl.BlockSpec((nb, din), lambda ph, i: (i * (1 - ph), 0)),
            pl.BlockSpec((nb, din), lambda ph, i: (i * (1 - ph), 0)),
            pl.BlockSpec(w1.shape, lambda ph, i: (0, 0)),
            pl.BlockSpec(b1.shape, lambda ph, i: (0, 0)),
            pl.BlockSpec(w2.shape, lambda ph, i: (0, 0)),
            pl.BlockSpec(b2.shape, lambda ph, i: (0, 0)),
            pl.BlockSpec((1, 1), lambda ph, i: (0, 0)),
            pl.BlockSpec((1, d2), lambda ph, i: (0, 0)),
            pl.BlockSpec((1, d2), lambda ph, i: (0, 0)),
        ],
        out_specs=pl.BlockSpec((nb, d2), lambda ph, i: (i, 0)),
        out_shape=jax.ShapeDtypeStruct((n, d2), jnp.float32),
        scratch_shapes=[
            pltpu.VMEM((n, d2), jnp.float32),
            pltpu.VMEM((8, d2), jnp.float32),
        ],
    )(xin, agg0, agg1, w1, b1, w2, b2, eps11, g, b)


def _pool_head(x1, x2, x3, x4, batch_row,
               w1, b1, w2, b2, w3, b3, w4, b4):
    n, d = x1.shape
    nb = 2000
    nblk = n // nb

    def body(b_ref, x1_ref, x2_ref, x3_ref, x4_ref,
             w1_ref, b1_ref, w2_ref, b2_ref, w3_ref, b3_ref, w4_ref, b4_ref,
             o_ref, acc_ref, cnt_ref):
        i = pl.program_id(0)

        @pl.when(i == 0)
        def _():
            acc_ref[...] = jnp.zeros_like(acc_ref)
            cnt_ref[...] = jnp.zeros_like(cnt_ref)

        seg = lax.broadcasted_iota(jnp.int32, (_NG, nb), 0)
        oh = (seg == b_ref[...].reshape(1, nb)).astype(jnp.float32)
        xcat = jnp.concatenate(
            [x1_ref[...], x2_ref[...], x3_ref[...], x4_ref[...]], axis=1)
        acc_ref[...] += lax.dot_general(
            oh, xcat, (((1,), (0,)), ((), ())),
            preferred_element_type=jnp.float32)
        cnt_ref[...] += jnp.sum(oh, axis=1, keepdims=True)

        @pl.when(i == nblk - 1)
        def _():
            pooled = acc_ref[...] / jnp.maximum(cnt_ref[...], 1.0)
            h1 = jnp.maximum(
                jnp.dot(pooled, w1_ref[...],
                        preferred_element_type=jnp.float32) + b1_ref[...], 0.0)
            h2 = jnp.maximum(
                jnp.dot(h1, w2_ref[...],
                        preferred_element_type=jnp.float32) + b2_ref[...], 0.0)
            h3 = jnp.maximum(
                jnp.dot(h2, w3_ref[...],
                        preferred_element_type=jnp.float32) + b3_ref[...], 0.0)
            z = (jnp.dot(h3, w4_ref[...],
                         preferred_element_type=jnp.float32) + b4_ref[...])
            m = jnp.max(z, axis=1, keepdims=True)
            lse = m + jnp.log(jnp.sum(jnp.exp(z - m), axis=1, keepdims=True))
            o_ref[...] = z - lse

    return pl.pallas_call(
        body,
        grid=(nblk,),
        in_specs=[
            pl.BlockSpec((1, 1, nb), lambda i: (i, 0, 0)),
            pl.BlockSpec((nb, d), lambda i: (i, 0)),
            pl.BlockSpec((nb, d), lambda i: (i, 0)),
            pl.BlockSpec((nb, d), lambda i: (i, 0)),
            pl.BlockSpec((nb, d), lambda i: (i, 0)),
            pl.BlockSpec(w1.shape, lambda i: (0, 0)),
            pl.BlockSpec(b1.shape, lambda i: (0, 0)),
            pl.BlockSpec(w2.shape, lambda i: (0, 0)),
            pl.BlockSpec(b2.shape, lambda i: (0, 0)),
            pl.BlockSpec(w3.shape, lambda i: (0, 0)),
            pl.BlockSpec(b3.shape, lambda i: (0, 0)),
            pl.BlockSpec(w4.shape, lambda i: (0, 0)),
            pl.BlockSpec(b4.shape, lambda i: (0, 0)),
        ],
        out_specs=pl.BlockSpec((_NG, 2), lambda i: (0, 0)),
        out_shape=jax.ShapeDtypeStruct((_NG, 2), jnp.float32),
        scratch_shapes=[
            pltpu.VMEM((_NG, 4 * d), jnp.float32),
            pltpu.VMEM((_NG, 1), jnp.float32),
        ],
    )(batch_row, x1, x2, x3, x4, w1, b1, w2, b2, w3, b3, w4, b4)


def _pad2(w, r, c):
    return jnp.pad(w, ((0, r - w.shape[0]), (0, c - w.shape[1])))


def _padb(b, c):
    return jnp.pad(b, (0, c - b.shape[0])).reshape(1, -1)


def kernel(x, edge_index, edge_attr, batch, params):
    p = params
    c_sz = 80
    eic = jnp.stack([edge_index[0].reshape(-1, c_sz),
                     edge_index[1].reshape(-1, c_sz)], axis=1)

    c1 = p["conv1"]
    ee1 = _edge_mlp(edge_attr,
                    _pad2(c1["be1"]["W"], 3, 16), _padb(c1["be1"]["b"], 16),
                    _pad2(c1["be2"]["W"], 16, 128), _padb(c1["be2"]["b"], 128))
    ees = [
        _edge_mlp(edge_attr, cv["be1"]["W"], cv["be1"]["b"].reshape(1, -1),
                  cv["be2"]["W"], cv["be2"]["b"].reshape(1, -1))
        for cv in (p["conv2"], p["conv3"], p["conv4"])
    ]

    x128 = jnp.pad(x, ((0, 0), (0, 128 - x.shape[1])))

    def layer(xin, cv, eew, bn, pad_in):
        ag = _sc_msgpass(xin, eew, eic)
        if pad_in:
            w1 = _pad2(cv["m1"]["W"], 128, 16)
            b1 = _padb(cv["m1"]["b"], 16)
            w2 = _pad2(cv["m2"]["W"], 16, 128)
        else:
            w1 = cv["m1"]["W"]
            b1 = cv["m1"]["b"].reshape(1, -1)
            w2 = cv["m2"]["W"]
        b2 = cv["m2"]["b"].reshape(1, -1)
        return _node_fused(xin, ag[0], ag[1], w1, b1, w2, b2,
                           cv["eps"].reshape(1, 1),
                           bn["g"].reshape(1, -1), bn["b"].reshape(1, -1))

    x1r = layer(x128, p["conv1"], ee1, p["bn1"], True)
    x2r = layer(x1r, p["conv2"], ees[0], p["bn2"], False)
    x3r = layer(x2r, p["conv3"], ees[1], p["bn3"], False)
    x4r = layer(x3r, p["conv4"], ees[2], p["bn4"], False)

    return _pool_head(
        x1r, x2r, x3r, x4r, batch.reshape(-1, 1, 2000),
        p["fc1"]["W"], p["fc1"]["b"].reshape(1, -1),
        p["fc2"]["W"], p["fc2"]["b"].reshape(1, -1),
        p["fc3"]["W"], p["fc3"]["b"].reshape(1, -1),
        p["fc4"]["W"], p["fc4"]["b"].reshape(1, -1))


# speedup vs baseline: 3.7744x; 1.4455x over previous
import functools

import jax
import jax.numpy as jnp
from jax import lax
from jax.experimental import pallas as pl
from jax.experimental.pallas import tpu as pltpu
from jax.experimental.pallas import tpu_sc as plsc

_NC, _NS = 2, 16
_NW = _NC * _NS
_NG = 64


def _edge_mlp(attr, w1, b1, w2, b2):
    e = attr.shape[0]
    dout = w2.shape[1]
    be = 4000

    def body(a_ref, w1_ref, b1_ref, w2_ref, b2_ref, o_ref):
        a = a_ref[...]
        h = jnp.maximum(
            jnp.dot(a, w1_ref[...], preferred_element_type=jnp.float32)
            + b1_ref[...], 0.0)
        o_ref[...] = (jnp.dot(h.astype(jnp.bfloat16),
                              w2_ref[...].astype(jnp.bfloat16),
                              preferred_element_type=jnp.float32)
                      + b2_ref[...])

    return pl.pallas_call(
        body,
        grid=(e // be,),
        in_specs=[
            pl.BlockSpec((be, attr.shape[1]), lambda i: (i, 0)),
            pl.BlockSpec(w1.shape, lambda i: (0, 0)),
            pl.BlockSpec(b1.shape, lambda i: (0, 0)),
            pl.BlockSpec(w2.shape, lambda i: (0, 0)),
            pl.BlockSpec(b2.shape, lambda i: (0, 0)),
        ],
        out_specs=pl.BlockSpec((be, dout), lambda i: (i, 0)),
        out_shape=jax.ShapeDtypeStruct((e, dout), jnp.float32),
    )(attr, w1, b1, w2, b2)


def _sc_msgpass(x, eew, eic):
    n, d = x.shape
    e = eic.shape[0] * eic.shape[2]
    ew = e // _NW
    c_sz = eic.shape[2]
    nch = ew // c_sz
    rpt, zc = 624, 8

    mesh = plsc.VectorSubcoreMesh(
        core_axis_name="c", subcore_axis_name="s",
        num_cores=_NC, num_subcores=_NS)

    ib = 32

    @functools.partial(
        pl.kernel,
        out_type=jax.ShapeDtypeStruct((_NC, n, d), jnp.float32),
        mesh=mesh,
        scratch_types=[
            pltpu.VMEM((ib, 2, c_sz), jnp.int32),
            pltpu.VMEM((2, c_sz, d), jnp.float32),
            pltpu.VMEM((2, c_sz, d), jnp.float32),
            pltpu.VMEM((zc, d), jnp.float32),
            pltpu.VMEM_SHARED((n, d), jnp.float32),
            [pltpu.SemaphoreType.DMA] * 2,
            [pltpu.SemaphoreType.DMA] * 2,
            pltpu.SemaphoreType.DMA,
        ],
    )
    def k(x_hbm, ee_hbm, ei_hbm, out_hbm,
          idx_v, ee_v, rows_v, zero_v, agg_sh, es, gs, dsem):
        ci = lax.axis_index("c")
        si = lax.axis_index("s")
        wid = si * _NC + ci

        def zrow(i, _):
            for j in range(d // 16):
                zero_v[i, pl.ds(j * 16, 16)] = jnp.zeros((16,), jnp.float32)
            return 0

        lax.fori_loop(0, zc, zrow, 0)
        for z in range(rpt // zc):
            pltpu.async_copy(zero_v,
                             agg_sh.at[pl.ds(si * rpt + z * zc, zc)], dsem)

        @pl.when(si == _NS - 1)
        def _():
            for q in range((n - _NS * rpt) // zc):
                pltpu.async_copy(
                    zero_v, agg_sh.at[pl.ds(_NS * rpt + q * zc, zc)], dsem)

        for z in range(rpt // zc):
            pltpu.make_async_copy(
                zero_v, agg_sh.at[pl.ds(z * zc, zc)], dsem).wait()

        @pl.when(si == _NS - 1)
        def _():
            for q in range((n - _NS * rpt) // zc):
                pltpu.make_async_copy(
                    zero_v, agg_sh.at[pl.ds(q * zc, zc)], dsem).wait()

        plsc.subcore_barrier()

        base = wid * ew
        base_g = wid * nch

        def start_fetch(s, row, b):
            pltpu.async_copy(
                ee_hbm.at[pl.ds(base + s * c_sz + row * c_sz, c_sz)],
                ee_v.at[b], es[b])
            pltpu.async_copy(x_hbm.at[idx_v.at[row, 0]], rows_v.at[b], gs[b])

        def finish_chunk(row, b):
            pltpu.make_async_copy(ee_hbm.at[pl.ds(0, c_sz)],
                                  ee_v.at[b], es[b]).wait()
            pltpu.make_async_copy(x_hbm.at[pl.ds(0, c_sz)],
                                  rows_v.at[b], gs[b]).wait()

            def msg(i, _):
                for j in range(d // 16):
                    sl = pl.ds(j * 16, 16)
                    rows_v[b, i, sl] = jnp.maximum(
                        rows_v[b, i, sl] + ee_v[b, i, sl], 0.0)
                return 0

            lax.fori_loop(0, c_sz, msg, 0)
            pltpu.sync_copy(rows_v.at[b],
                            agg_sh.at[idx_v.at[row, 1]], add=True)

        def run_phase(s, m):
            start_fetch(s, 0, 0)

            def pair(i, _):
                r0 = i * 2
                for db in (0, 1):
                    start_fetch(s, r0 + db + 1, 1 - db)
                    finish_chunk(r0 + db, db)
                return 0

            lax.fori_loop(0, (m - 1) // 2, pair, 0)
            if (m - 1) % 2 == 0:
                finish_chunk(m - 1, (m - 1) % 2)
            else:
                start_fetch(s, m - 1, (m - 1) % 2)
                finish_chunk(m - 2, (m - 2) % 2)
                finish_chunk(m - 1, (m - 1) % 2)

        done = 0
        while done < nch:
            m = min(ib, nch - done)
            pltpu.sync_copy(ei_hbm.at[pl.ds(base_g + done, m)],
                            idx_v.at[pl.ds(0, m)])
            run_phase(done, m)
            done += m

        plsc.subcore_barrier()

        for z in range(rpt // 208):
            row0 = si * rpt + z * 208
            pltpu.sync_copy(agg_sh.at[pl.ds(row0, 208)],
                            out_hbm.at[ci, pl.ds(row0, 208)])

        @pl.when(si == _NS - 1)
        def _():
            pltpu.sync_copy(agg_sh.at[pl.ds(_NS * rpt, n - _NS * rpt)],
                            out_hbm.at[ci, pl.ds(_NS * rpt, n - _NS * rpt)])

    return k(x, eew, eic)


def _node_fused(xin, agg0, agg1, w1, b1, w2, b2, eps11, g, b):
    n, din = xin.shape
    d2 = w2.shape[1]
    nb = 1000
    nblk = n // nb

    def body(x_ref, a0_ref, a1_ref, w1_ref, b1_ref, w2_ref, b2_ref, eps_ref,
             g_ref, b_ref, o_ref, y_ref, st_ref):
        ph = pl.program_id(0)
        i = pl.program_id(1)

        @pl.when(ph == 0)
        def _():
            h = (x_ref[...] * (1.0 + eps_ref[0, 0])
                 + a0_ref[...] + a1_ref[...])
            t = jnp.maximum(
                jnp.dot(h, w1_ref[...], preferred_element_type=jnp.float32)
                + b1_ref[...], 0.0)
            y = jnp.maximum(
                jnp.dot(t, w2_ref[...], preferred_element_type=jnp.float32)
                + b2_ref[...], 0.0)
            y_ref[pl.ds(i * nb, nb), :] = y

            @pl.when(i == 0)
            def _():
                st_ref[...] = jnp.zeros_like(st_ref)

            st_ref[0:1, :] += jnp.sum(y, axis=0, keepdims=True)
            st_ref[1:2, :] += jnp.sum(y * y, axis=0, keepdims=True)

        @pl.when(ph == 1)
        def _():
            mean = st_ref[0:1, :] / n
            var = st_ref[1:2, :] / n - mean * mean
            o_ref[...] = ((y_ref[pl.ds(i * nb, nb), :] - mean)
                          * lax.rsqrt(var + 1e-5) * g_ref[...] + b_ref[...])

    return pl.pallas_call(
        body,
        grid=(2, nblk),
        in_specs=[
            pl.BlockSpec((nb, din), lambda ph, i: (i * (1 - ph), 0)),
            pl.BlockSpec((nb, din), lambda ph, i: (i * (1 - ph), 0)),
            pl.BlockSpec((nb, din), lambda ph, i: (i * (1 - ph), 0)),
            pl.BlockSpec(w1.shape, lambda ph, i: (0, 0)),
            pl.BlockSpec(b1.shape, lambda ph, i: (0, 0)),
            pl.BlockSpec(w2.shape, lambda ph, i: (0, 0)),
            pl.BlockSpec(b2.shape, lambda ph, i: (0, 0)),
            pl.BlockSpec((1, 1), lambda ph, i: (0, 0)),
            pl.BlockSpec((1, d2), lambda ph, i: (0, 0)),
            pl.BlockSpec((1, d2), lambda ph, i: (0, 0)),
        ],
        out_specs=pl.BlockSpec((nb, d2), lambda ph, i: (i, 0)),
        out_shape=jax.ShapeDtypeStruct((n, d2), jnp.float32),
        scratch_shapes=[
            pltpu.VMEM((n, d2), jnp.float32),
            pltpu.VMEM((8, d2), jnp.float32),
        ],
    )(xin, agg0, agg1, w1, b1, w2, b2, eps11, g, b)


def _pool_head(x1, x2, x3, x4, batch_row,
               w1, b1, w2, b2, w3, b3, w4, b4):
    n, d = x1.shape
    nb = 2000
    nblk = n // nb

    def body(b_ref, x1_ref, x2_ref, x3_ref, x4_ref,
             w1_ref, b1_ref, w2_ref, b2_ref, w3_ref, b3_ref, w4_ref, b4_ref,
             o_ref, acc_ref, cnt_ref):
        i = pl.program_id(0)

        @pl.when(i == 0)
        def _():
            acc_ref[...] = jnp.zeros_like(acc_ref)
            cnt_ref[...] = jnp.zeros_like(cnt_ref)

        seg = lax.broadcasted_iota(jnp.int32, (_NG, nb), 0)
        oh = (seg == b_ref[...].reshape(1, nb)).astype(jnp.float32)
        xcat = jnp.concatenate(
            [x1_ref[...], x2_ref[...], x3_ref[...], x4_ref[...]], axis=1)
        acc_ref[...] += lax.dot_general(
            oh, xcat, (((1,), (0,)), ((), ())),
            preferred_element_type=jnp.float32)
        cnt_ref[...] += jnp.sum(oh, axis=1, keepdims=True)

        @pl.when(i == nblk - 1)
        def _():
            pooled = acc_ref[...] / jnp.maximum(cnt_ref[...], 1.0)
            h1 = jnp.maximum(
                jnp.dot(pooled, w1_ref[...],
                        preferred_element_type=jnp.float32) + b1_ref[...], 0.0)
            h2 = jnp.maximum(
                jnp.dot(h1, w2_ref[...],
                        preferred_element_type=jnp.float32) + b2_ref[...], 0.0)
            h3 = jnp.maximum(
                jnp.dot(h2, w3_ref[...],
                        preferred_element_type=jnp.float32) + b3_ref[...], 0.0)
            z = (jnp.dot(h3, w4_ref[...],
                         preferred_element_type=jnp.float32) + b4_ref[...])
            m = jnp.max(z, axis=1, keepdims=True)
            lse = m + jnp.log(jnp.sum(jnp.exp(z - m), axis=1, keepdims=True))
            o_ref[...] = z - lse

    return pl.pallas_call(
        body,
        grid=(nblk,),
        in_specs=[
            pl.BlockSpec((1, 1, nb), lambda i: (i, 0, 0)),
            pl.BlockSpec((nb, d), lambda i: (i, 0)),
            pl.BlockSpec((nb, d), lambda i: (i, 0)),
            pl.BlockSpec((nb, d), lambda i: (i, 0)),
            pl.BlockSpec((nb, d), lambda i: (i, 0)),
            pl.BlockSpec(w1.shape, lambda i: (0, 0)),
            pl.BlockSpec(b1.shape, lambda i: (0, 0)),
            pl.BlockSpec(w2.shape, lambda i: (0, 0)),
            pl.BlockSpec(b2.shape, lambda i: (0, 0)),
            pl.BlockSpec(w3.shape, lambda i: (0, 0)),
            pl.BlockSpec(b3.shape, lambda i: (0, 0)),
            pl.BlockSpec(w4.shape, lambda i: (0, 0)),
            pl.BlockSpec(b4.shape, lambda i: (0, 0)),
        ],
        out_specs=pl.BlockSpec((_NG, 2), lambda i: (0, 0)),
        out_shape=jax.ShapeDtypeStruct((_NG, 2), jnp.float32),
        scratch_shapes=[
            pltpu.VMEM((_NG, 4 * d), jnp.float32),
            pltpu.VMEM((_NG, 1), jnp.float32),
        ],
    )(batch_row, x1, x2, x3, x4, w1, b1, w2, b2, w3, b3, w4, b4)


def _pad2(w, r, c):
    return jnp.pad(w, ((0, r - w.shape[0]), (0, c - w.shape[1])))


def _padb(b, c):
    return jnp.pad(b, (0, c - b.shape[0])).reshape(1, -1)


def kernel(x, edge_index, edge_attr, batch, params):
    p = params
    c_sz = 80
    eic = jnp.stack([edge_index[0].reshape(-1, c_sz),
                     edge_index[1].reshape(-1, c_sz)], axis=1)

    c1 = p["conv1"]
    ee1 = _edge_mlp(edge_attr,
                    _pad2(c1["be1"]["W"], 3, 16), _padb(c1["be1"]["b"], 16),
                    _pad2(c1["be2"]["W"], 16, 128), _padb(c1["be2"]["b"], 128))
    ees = [
        _edge_mlp(edge_attr, cv["be1"]["W"], cv["be1"]["b"].reshape(1, -1),
                  cv["be2"]["W"], cv["be2"]["b"].reshape(1, -1))
        for cv in (p["conv2"], p["conv3"], p["conv4"])
    ]

    x128 = jnp.pad(x, ((0, 0), (0, 128 - x.shape[1])))

    def layer(xin, cv, eew, bn, pad_in):
        ag = _sc_msgpass(xin, eew, eic)
        if pad_in:
            w1 = _pad2(cv["m1"]["W"], 128, 16)
            b1 = _padb(cv["m1"]["b"], 16)
            w2 = _pad2(cv["m2"]["W"], 16, 128)
        else:
            w1 = cv["m1"]["W"]
            b1 = cv["m1"]["b"].reshape(1, -1)
            w2 = cv["m2"]["W"]
        b2 = cv["m2"]["b"].reshape(1, -1)
        return _node_fused(xin, ag[0], ag[1], w1, b1, w2, b2,
                           cv["eps"].reshape(1, 1),
                           bn["g"].reshape(1, -1), bn["b"].reshape(1, -1))

    x1r = layer(x128, p["conv1"], ee1, p["bn1"], True)
    x2r = layer(x1r, p["conv2"], ees[0], p["bn2"], False)
    x3r = layer(x2r, p["conv3"], ees[1], p["bn3"], False)
    x4r = layer(x3r, p["conv4"], ees[2], p["bn4"], False)

    return _pool_head(
        x1r, x2r, x3r, x4r, batch.reshape(-1, 1, 2000),
        p["fc1"]["W"], p["fc1"]["b"].reshape(1, -1),
        p["fc2"]["W"], p["fc2"]["b"].reshape(1, -1),
        p["fc3"]["W"], p["fc3"]["b"].reshape(1, -1),
        p["fc4"]["W"], p["fc4"]["b"].reshape(1, -1))
